# baseline (speedup 1.0000x reference)
import jax
import jax.numpy as jnp
from jax.experimental import pallas as pl
from jax.experimental.pallas import tpu as pltpu

_N = 16384
_DIM = 2048
_E = 64
_TILE = 512


def _gate_tile(x_ref, w_ref, b_ref, vals_ref, idx_ref):
    x = x_ref[...]
    w = w_ref[...]
    logits = jax.lax.dot_general(
        w, x, (((1,), (1,)), ((), ())), preferred_element_type=jnp.float32)
    logits = logits + b_ref[...]

    rowf = jax.lax.broadcasted_iota(
        jnp.int32, logits.shape, 0).astype(jnp.float32)

    m1 = jnp.max(logits, axis=0, keepdims=True)
    i1f = jnp.min(jnp.where(logits == m1, rowf, float(_E)),
                  axis=0, keepdims=True)

    masked = jnp.where(rowf == i1f, -jnp.inf, logits)
    m2 = jnp.max(masked, axis=0, keepdims=True)
    i2f = jnp.min(jnp.where(masked == m2, rowf, float(_E)),
                  axis=0, keepdims=True)

    denom = jnp.sum(jnp.exp(logits - m1), axis=0, keepdims=True)
    v1 = 1.0 / denom
    v2 = jnp.exp(m2 - m1) * v1

    vals_ref[...] = jnp.concatenate([v1, v2], axis=0)
    idx_ref[...] = jnp.concatenate([i1f, i2f], axis=0).astype(jnp.int32)


def kernel(x, weight, bias):
    n = x.shape[0]
    grid = (n // _TILE,)
    vals_t, idx_t = pl.pallas_call(
        _gate_tile,
        grid=grid,
        in_specs=[
            pl.BlockSpec((_TILE, _DIM), lambda i: (i, 0)),
            pl.BlockSpec((_E, _DIM), lambda i: (0, 0)),
            pl.BlockSpec((_E, 1), lambda i: (0, 0)),
        ],
        out_specs=[
            pl.BlockSpec((2, _TILE), lambda i: (0, i)),
            pl.BlockSpec((2, _TILE), lambda i: (0, i)),
        ],
        out_shape=[
            jax.ShapeDtypeStruct((2, n), jnp.float32),
            jax.ShapeDtypeStruct((2, n), jnp.int32),
        ],
        compiler_params=pltpu.CompilerParams(
            dimension_semantics=("arbitrary",)),
    )(x, weight, bias.reshape(_E, 1))
    return vals_t.T, idx_t.T


# speedup vs baseline: 1.1095x; 1.1095x over previous
import jax
import jax.numpy as jnp
from jax.experimental import pallas as pl
from jax.experimental.pallas import tpu as pltpu

_N = 16384
_DIM = 2048
_E = 64
_TILE = 1024


def _gate_tile(x_ref, w_ref, b_ref, out_ref):
    x = x_ref[...]
    w = w_ref[...]
    logits = jax.lax.dot_general(
        w, x, (((1,), (1,)), ((), ())), preferred_element_type=jnp.float32)
    logits = logits + b_ref[...]

    rowf = jax.lax.broadcasted_iota(
        jnp.int32, logits.shape, 0).astype(jnp.float32)

    m1 = jnp.max(logits, axis=0, keepdims=True)
    i1f = jnp.min(jnp.where(logits == m1, rowf, float(_E)),
                  axis=0, keepdims=True)

    masked = jnp.where(rowf == i1f, -jnp.inf, logits)
    m2 = jnp.max(masked, axis=0, keepdims=True)
    i2f = jnp.min(jnp.where(masked == m2, rowf, float(_E)),
                  axis=0, keepdims=True)

    denom = jnp.sum(jnp.exp(logits - m1), axis=0, keepdims=True)
    v1 = 1.0 / denom
    v2 = jnp.exp(m2 - m1) * v1

    idx_bits = jax.lax.bitcast_convert_type(
        jnp.concatenate([i1f, i2f], axis=0).astype(jnp.int32), jnp.float32)
    out_ref[...] = jnp.concatenate([v1, v2, idx_bits], axis=0)


def kernel(x, weight, bias):
    n = x.shape[0]
    grid = (n // _TILE,)
    out = pl.pallas_call(
        _gate_tile,
        grid=grid,
        in_specs=[
            pl.BlockSpec((_TILE, _DIM), lambda i: (i, 0)),
            pl.BlockSpec((_E, _DIM), lambda i: (0, 0)),
            pl.BlockSpec((_E, 1), lambda i: (0, 0)),
        ],
        out_specs=pl.BlockSpec((4, _TILE), lambda i: (0, i)),
        out_shape=jax.ShapeDtypeStruct((4, n), jnp.float32),
        compiler_params=pltpu.CompilerParams(
            dimension_semantics=("arbitrary",)),
    )(x, weight, bias.reshape(_E, 1))
    vals = out[:2].T
    idx = jax.lax.bitcast_convert_type(out[2:], jnp.int32).T
    return vals, idx


# speedup vs baseline: 1.1912x; 1.0736x over previous
import jax
import jax.numpy as jnp
from jax.experimental import pallas as pl
from jax.experimental.pallas import tpu as pltpu

_N = 16384
_DIM = 2048
_E = 64
_TILE = 1024


def _gate_tile(x_ref, w_ref, b_ref, vals_ref, idx_ref):
    x = x_ref[...]
    w = w_ref[...]
    logits = jax.lax.dot_general(
        w, x, (((1,), (1,)), ((), ())), preferred_element_type=jnp.float32)
    logits = logits + b_ref[...]

    rowf = jax.lax.broadcasted_iota(
        jnp.int32, logits.shape, 0).astype(jnp.float32)

    m1 = jnp.max(logits, axis=0, keepdims=True)
    i1f = jnp.min(jnp.where(logits == m1, rowf, float(_E)),
                  axis=0, keepdims=True)

    masked = jnp.where(rowf == i1f, -jnp.inf, logits)
    m2 = jnp.max(masked, axis=0, keepdims=True)
    i2f = jnp.min(jnp.where(masked == m2, rowf, float(_E)),
                  axis=0, keepdims=True)

    denom = jnp.sum(jnp.exp(logits - m1), axis=0, keepdims=True)
    v1 = 1.0 / denom
    v2 = jnp.exp(m2 - m1) * v1

    vals_ref[...] = jnp.concatenate([v1, v2], axis=0)
    idx_ref[...] = jnp.concatenate([i1f, i2f], axis=0).astype(jnp.int32)


def kernel(x, weight, bias):
    n = x.shape[0]
    grid = (n // _TILE,)
    vals_t, idx_t = pl.pallas_call(
        _gate_tile,
        grid=grid,
        in_specs=[
            pl.BlockSpec((_TILE, _DIM), lambda i: (i, 0)),
            pl.BlockSpec((_E, _DIM), lambda i: (0, 0)),
            pl.BlockSpec((_E, 1), lambda i: (0, 0)),
        ],
        out_specs=[
            pl.BlockSpec((2, _TILE), lambda i: (0, i)),
            pl.BlockSpec((2, _TILE), lambda i: (0, i)),
        ],
        out_shape=[
            jax.ShapeDtypeStruct((2, n), jnp.float32),
            jax.ShapeDtypeStruct((2, n), jnp.int32),
        ],
        compiler_params=pltpu.CompilerParams(
            dimension_semantics=("parallel",)),
    )(x, weight, bias.reshape(_E, 1))
    return vals_t.T, idx_t.T


# speedup vs baseline: 1.1937x; 1.0021x over previous
import jax
import jax.numpy as jnp
from jax.experimental import pallas as pl
from jax.experimental.pallas import tpu as pltpu

_N = 16384
_DIM = 2048
_E = 64
_TILE = 2048


def _gate_tile(x_ref, w_ref, b_ref, vals_ref, idx_ref):
    x = x_ref[...]
    w = w_ref[...]
    logits = jax.lax.dot_general(
        w, x, (((1,), (1,)), ((), ())), preferred_element_type=jnp.float32)
    logits = logits + b_ref[...]

    rowf = jax.lax.broadcasted_iota(
        jnp.int32, logits.shape, 0).astype(jnp.float32)

    m1 = jnp.max(logits, axis=0, keepdims=True)
    i1f = jnp.min(jnp.where(logits == m1, rowf, float(_E)),
                  axis=0, keepdims=True)

    masked = jnp.where(rowf == i1f, -jnp.inf, logits)
    m2 = jnp.max(masked, axis=0, keepdims=True)
    i2f = jnp.min(jnp.where(masked == m2, rowf, float(_E)),
                  axis=0, keepdims=True)

    denom = jnp.sum(jnp.exp(logits - m1), axis=0, keepdims=True)
    v1 = 1.0 / denom
    v2 = jnp.exp(m2 - m1) * v1

    vals_ref[...] = jnp.concatenate([v1, v2], axis=0)
    idx_ref[...] = jnp.concatenate([i1f, i2f], axis=0).astype(jnp.int32)


def kernel(x, weight, bias):
    n = x.shape[0]
    grid = (n // _TILE,)
    vals_t, idx_t = pl.pallas_call(
        _gate_tile,
        grid=grid,
        in_specs=[
            pl.BlockSpec((_TILE, _DIM), lambda i: (i, 0)),
            pl.BlockSpec((_E, _DIM), lambda i: (0, 0)),
            pl.BlockSpec((_E, 1), lambda i: (0, 0)),
        ],
        out_specs=[
            pl.BlockSpec((2, _TILE), lambda i: (0, i)),
            pl.BlockSpec((2, _TILE), lambda i: (0, i)),
        ],
        out_shape=[
            jax.ShapeDtypeStruct((2, n), jnp.float32),
            jax.ShapeDtypeStruct((2, n), jnp.int32),
        ],
        compiler_params=pltpu.CompilerParams(
            dimension_semantics=("parallel",)),
    )(x, weight, bias.reshape(_E, 1))
    return vals_t.T, idx_t.T
